# baseline (speedup 1.0000x reference)
import jax
import jax.numpy as jnp
from jax import lax
from jax.experimental import pallas as pl
from jax.experimental.pallas import tpu as pltpu
from jax.experimental.pallas import tpu_sc as plsc

N = 10000
E = 320000
D = 128
DE = 16
H = 128
L = 3
C = 40

NC = 2
NS = 16
NW = NC * NS
CHUNK = 128
NCHUNKS = E // CHUNK
CPW = NCHUNKS // NW
NTAIL = NCHUNKS - CPW * NW
CPW2 = 80
HCPW = 40
NCHUNKS2 = CPW2 * NW
EPAD = NCHUNKS2 * CHUNK
NPAD = N + 16
RPT = 624
RTAIL = N - RPT * NS
BE = 8000


def _tile_copy(s, src_ref, dst_ref):
    row0 = s * RPT
    pltpu.sync_copy(src_ref.at[pl.ds(row0, RPT), :],
                    dst_ref.at[pl.ds(row0, RPT), :])

    @pl.when(s == NS - 1)
    def _():
        pltpu.sync_copy(src_ref.at[pl.ds(RPT * NS, RTAIL), :],
                        dst_ref.at[pl.ds(RPT * NS, RTAIL), :])


def _spmv_body(table, src2, dst2, zinit, out,
               sidx, didx, rows_v0, rows_v1, acc, sem0, sem1):
    c = lax.axis_index("c")
    s = lax.axis_index("s")
    wid = c * NS + s

    _tile_copy(s, zinit, acc)
    j0 = wid * CPW2
    plsc.subcore_barrier()

    rv = (rows_v0, rows_v1)
    sems = (sem0, sem1)

    def start(k, b):
        pltpu.async_copy(table.at[sidx.at[k]], rv[b], sems[b])

    def finish(k, b):
        pltpu.make_async_copy(table.at[sidx.at[k]], rv[b], sems[b]).wait()
        pltpu.sync_copy(rv[b], acc.at[didx.at[k]], add=True)

    for hf in range(CPW2 // HCPW):
        pltpu.sync_copy(src2.at[pl.ds(j0 + hf * HCPW, HCPW), :], sidx)
        pltpu.sync_copy(dst2.at[pl.ds(j0 + hf * HCPW, HCPW), :], didx)
        start(0, 0)
        start(1, 1)

        def body(i, _):
            k = 2 * i
            finish(k, 0)
            start(k + 2, 0)
            finish(k + 1, 1)
            start(k + 3, 1)
            return 0

        lax.fori_loop(0, (HCPW - 2) // 2, body, 0)
        finish(HCPW - 2, 0)
        finish(HCPW - 1, 1)

    plsc.subcore_barrier()
    _tile_copy(s, acc, out.at[c])


_sc_spmv = pl.kernel(
    _spmv_body,
    out_type=jax.ShapeDtypeStruct((NC, N, D), jnp.float32),
    mesh=plsc.VectorSubcoreMesh(core_axis_name="c", subcore_axis_name="s"),
    scratch_types=[
        pltpu.VMEM((HCPW, CHUNK), jnp.int32),
        pltpu.VMEM((HCPW, CHUNK), jnp.int32),
        pltpu.VMEM((CHUNK, D), jnp.float32),
        pltpu.VMEM((CHUNK, D), jnp.float32),
        pltpu.VMEM_SHARED((NPAD, D), jnp.float32),
        pltpu.SemaphoreType.DMA,
        pltpu.SemaphoreType.DMA,
    ],
)


def _edge_scatter_body(eh_full, dst, zinit, out,
                       dst_v0, rows_v0, dst_v1, rows_v1, acc, sem0, sem1):
    c = lax.axis_index("c")
    s = lax.axis_index("s")
    wid = c * NS + s

    _tile_copy(s, zinit, acc)
    plsc.subcore_barrier()

    dv = (dst_v0, dst_v1)
    rv = (rows_v0, rows_v1)
    sems = (sem0, sem1)

    def start(j, b):
        base = j * CHUNK
        pltpu.sync_copy(dst.at[pl.ds(base, CHUNK)], dv[b])
        pltpu.async_copy(eh_full.at[pl.ds(base, CHUNK), :], rv[b], sems[b])

    def finish(j, b):
        base = j * CHUNK
        pltpu.make_async_copy(eh_full.at[pl.ds(base, CHUNK), :], rv[b],
                              sems[b]).wait()
        pltpu.sync_copy(rv[b], acc.at[dv[b]], add=True)

    j0 = wid * CPW
    start(j0, 0)
    start(j0 + 1, 1)

    def body(i, _):
        j = j0 + 2 * i
        finish(j, 0)
        start(j + 2, 0)
        finish(j + 1, 1)
        start(j + 3, 1)
        return 0

    lax.fori_loop(0, (CPW - 2) // 2, body, 0)
    finish(j0 + CPW - 2, 0)
    finish(j0 + CPW - 1, 1)

    @pl.when(wid < NTAIL)
    def _():
        start(NW * CPW + wid, 0)
        finish(NW * CPW + wid, 0)

    plsc.subcore_barrier()
    _tile_copy(s, acc, out.at[c])


_sc_edge_scatter = pl.kernel(
    _edge_scatter_body,
    out_type=jax.ShapeDtypeStruct((NC, N, D), jnp.float32),
    mesh=plsc.VectorSubcoreMesh(core_axis_name="c", subcore_axis_name="s"),
    scratch_types=[
        pltpu.VMEM((CHUNK,), jnp.int32),
        pltpu.VMEM((CHUNK, D), jnp.float32),
        pltpu.VMEM((CHUNK,), jnp.int32),
        pltpu.VMEM((CHUNK, D), jnp.float32),
        pltpu.VMEM_SHARED((N, D), jnp.float32),
        pltpu.SemaphoreType.DMA,
        pltpu.SemaphoreType.DMA,
    ],
)


_DN = (((1,), (1,)), ((), ()))


def _eh_body(e_ref, we_ref, be_ref, o_ref):
    o_ref[...] = lax.dot_general(e_ref[...], we_ref[...], _DN,
                                 preferred_element_type=jnp.float32) + be_ref[...]


def _tc_eh(e, W_edge, b_edge):
    return pl.pallas_call(
        _eh_body,
        grid=(E // BE,),
        in_specs=[pl.BlockSpec((BE, DE), lambda i: (i, 0)),
                  pl.BlockSpec((H, DE), lambda i: (0, 0)),
                  pl.BlockSpec((1, H), lambda i: (0, 0))],
        out_specs=pl.BlockSpec((BE, H), lambda i: (i, 0)),
        out_shape=jax.ShapeDtypeStruct((E, H), jnp.float32),
    )(e, W_edge, b_edge.reshape(1, H))


def _prep_body(x_ref, wn_ref, bn_ref, pe_ref, h_ref, eh_ref):
    h_ref[...] = lax.dot_general(x_ref[...], wn_ref[...], _DN,
                                 preferred_element_type=jnp.float32) + bn_ref[...]
    eh_ref[...] = pe_ref[0] + pe_ref[1]


def _tc_prep(x, W_node, b_node, pe):
    return pl.pallas_call(
        _prep_body,
        out_shape=[jax.ShapeDtypeStruct((N, H), jnp.float32),
                   jax.ShapeDtypeStruct((N, H), jnp.float32)],
    )(x, W_node, b_node.reshape(1, H), pe)


def _layer_body(p_ref, eh_ref, tin_ref, w1_ref, b1_ref, g1_ref, be1_ref,
                w2_ref, b2_ref, hs_ref, tout_ref):
    aggr = p_ref[0] + p_ref[1] + eh_ref[...]
    z = lax.dot_general(aggr, w1_ref[...], _DN,
                        preferred_element_type=jnp.float32) + b1_ref[...]
    m = jnp.mean(z, axis=0, keepdims=True)
    v = jnp.mean((z - m) ** 2, axis=0, keepdims=True)
    z = g1_ref[...] * (z - m) / jnp.sqrt(v + 1e-5) + be1_ref[...]
    z = jnp.maximum(z, 0.0)
    z = lax.dot_general(z, w2_ref[...], _DN,
                        preferred_element_type=jnp.float32) + b2_ref[...]
    hs = jnp.where(z > 0, z, 0.2 * z)
    hs_ref[...] = hs
    tout_ref[...] = tin_ref[...] + hs


def _tc_layer(p, eh, total, W1l, b1l, g1l, be1l, W2l, b2l):
    return pl.pallas_call(
        _layer_body,
        out_shape=[jax.ShapeDtypeStruct((N, H), jnp.float32),
                   jax.ShapeDtypeStruct((N, H), jnp.float32)],
    )(p, eh, total, W1l, b1l.reshape(1, 2 * H), g1l.reshape(1, 2 * H),
      be1l.reshape(1, 2 * H), W2l, b2l.reshape(1, H))


def _head_body(t_ref, wo1_ref, bo1_ref, go_ref, beo_ref, ap_ref, wo2_ref,
               bo2_ref, o_ref):
    z = lax.dot_general(t_ref[...], wo1_ref[...], _DN,
                        preferred_element_type=jnp.float32) + bo1_ref[...]
    m = jnp.mean(z, axis=0, keepdims=True)
    v = jnp.mean((z - m) ** 2, axis=0, keepdims=True)
    z = go_ref[...] * (z - m) / jnp.sqrt(v + 1e-5) + beo_ref[...]
    z = jnp.where(z > 0, z, ap_ref[...] * z)
    o_ref[...] = lax.dot_general(z, wo2_ref[...], _DN,
                                 preferred_element_type=jnp.float32) + bo2_ref[...]


def _tc_head(total, Wo1, bo1, go, beo, a_prelu, Wo2, bo2):
    return pl.pallas_call(
        _head_body,
        out_shape=jax.ShapeDtypeStruct((N, C), jnp.float32),
    )(total, Wo1, bo1.reshape(1, 2 * H), go.reshape(1, 2 * H),
      beo.reshape(1, 2 * H), a_prelu.reshape(1, 1), Wo2, bo2.reshape(1, C))


def kernel(x, edge_index, e, W_node, b_node, W_edge, b_edge, W1, b1, g1, be1,
           W2, b2, Wo1, bo1, go, beo, a_prelu, Wo2, bo2):
    src = edge_index[0]
    dst = edge_index[1]

    pad = EPAD - E
    src2 = jnp.concatenate([src, jnp.zeros((pad,), jnp.int32)]).reshape(
        NCHUNKS2, CHUNK)
    dst2 = jnp.concatenate([dst, jnp.full((pad,), N, jnp.int32)]).reshape(
        NCHUNKS2, CHUNK)

    zin_d = jnp.zeros((N, D), jnp.float32)

    eh_full = _tc_eh(e, W_edge, b_edge)
    pe = _sc_edge_scatter(eh_full, dst, zin_d)
    h, eh = _tc_prep(x, W_node, b_node, pe)

    hs = h
    total = h
    for l in range(L):
        p = _sc_spmv(hs, src2, dst2, zin_d)
        hs, total = _tc_layer(p, eh, total, W1[l], b1[l], g1[l], be1[l],
                              W2[l], b2[l])

    return _tc_head(total, Wo1, bo1, go, beo, a_prelu, Wo2, bo2)


# speedup vs baseline: 1.0001x; 1.0001x over previous
import jax
import jax.numpy as jnp
from jax import lax
from jax.experimental import pallas as pl
from jax.experimental.pallas import tpu as pltpu
from jax.experimental.pallas import tpu_sc as plsc

N = 10000
E = 320000
D = 128
DE = 16
H = 128
L = 3
C = 40

NC = 2
NS = 16
NW = NC * NS
CHUNK = 128
NCHUNKS = E // CHUNK
CPW = NCHUNKS // NW
NTAIL = NCHUNKS - CPW * NW
CPW2 = 80
HCPW = 40
NCHUNKS2 = CPW2 * NW
EPAD = NCHUNKS2 * CHUNK
NPAD = N + 16
RPT = 624
RTAIL = N - RPT * NS
BE = 8000


def _tile_copy(s, src_ref, dst_ref):
    row0 = s * RPT
    pltpu.sync_copy(src_ref.at[pl.ds(row0, RPT), :],
                    dst_ref.at[pl.ds(row0, RPT), :])

    @pl.when(s == NS - 1)
    def _():
        pltpu.sync_copy(src_ref.at[pl.ds(RPT * NS, RTAIL), :],
                        dst_ref.at[pl.ds(RPT * NS, RTAIL), :])


def _spmv_body(table, src2, dst2, zinit, out,
               sidx, didx, rows_v0, rows_v1, acc, sem0, sem1):
    c = lax.axis_index("c")
    s = lax.axis_index("s")
    wid = c * NS + s

    _tile_copy(s, zinit, acc)
    j0 = wid * CPW2
    plsc.subcore_barrier()

    rv = (rows_v0, rows_v1)
    sems = (sem0, sem1)

    def start(k, b):
        pltpu.async_copy(table.at[sidx.at[k]], rv[b], sems[b])

    def finish(k, b):
        pltpu.make_async_copy(table.at[sidx.at[k]], rv[b], sems[b]).wait()
        pltpu.sync_copy(rv[b], acc.at[didx.at[k]], add=True)

    for hf in range(CPW2 // HCPW):
        pltpu.sync_copy(src2.at[pl.ds(j0 + hf * HCPW, HCPW), :], sidx)
        pltpu.sync_copy(dst2.at[pl.ds(j0 + hf * HCPW, HCPW), :], didx)
        start(0, 0)
        start(1, 1)

        def body(i, _):
            k = 2 * i
            finish(k, 0)
            start(k + 2, 0)
            finish(k + 1, 1)
            start(k + 3, 1)
            return 0

        lax.fori_loop(0, (HCPW - 2) // 2, body, 0)
        finish(HCPW - 2, 0)
        finish(HCPW - 1, 1)

    plsc.subcore_barrier()
    _tile_copy(s, acc, out.at[c])


_sc_spmv = pl.kernel(
    _spmv_body,
    out_type=jax.ShapeDtypeStruct((NC, N, D), jnp.float32),
    mesh=plsc.VectorSubcoreMesh(core_axis_name="c", subcore_axis_name="s"),
    scratch_types=[
        pltpu.VMEM((HCPW, CHUNK), jnp.int32),
        pltpu.VMEM((HCPW, CHUNK), jnp.int32),
        pltpu.VMEM((CHUNK, D), jnp.float32),
        pltpu.VMEM((CHUNK, D), jnp.float32),
        pltpu.VMEM_SHARED((NPAD, D), jnp.float32),
        pltpu.SemaphoreType.DMA,
        pltpu.SemaphoreType.DMA,
    ],
)


def _edge_scatter_body(eh_full, dst, zinit, out,
                       dst_v0, rows_v0, dst_v1, rows_v1, acc, sem0, sem1):
    c = lax.axis_index("c")
    s = lax.axis_index("s")
    wid = c * NS + s

    _tile_copy(s, zinit, acc)
    plsc.subcore_barrier()

    dv = (dst_v0, dst_v1)
    rv = (rows_v0, rows_v1)
    sems = (sem0, sem1)

    def start(j, b):
        base = j * CHUNK
        pltpu.sync_copy(dst.at[pl.ds(base, CHUNK)], dv[b])
        pltpu.async_copy(eh_full.at[pl.ds(base, CHUNK), :], rv[b], sems[b])

    def finish(j, b):
        base = j * CHUNK
        pltpu.make_async_copy(eh_full.at[pl.ds(base, CHUNK), :], rv[b],
                              sems[b]).wait()
        pltpu.sync_copy(rv[b], acc.at[dv[b]], add=True)

    j0 = wid * CPW
    start(j0, 0)
    start(j0 + 1, 1)

    def body(i, _):
        j = j0 + 2 * i
        finish(j, 0)
        start(j + 2, 0)
        finish(j + 1, 1)
        start(j + 3, 1)
        return 0

    lax.fori_loop(0, (CPW - 2) // 2, body, 0)
    finish(j0 + CPW - 2, 0)
    finish(j0 + CPW - 1, 1)

    @pl.when(wid < NTAIL)
    def _():
        start(NW * CPW + wid, 0)
        finish(NW * CPW + wid, 0)

    plsc.subcore_barrier()
    _tile_copy(s, acc, out.at[c])


_sc_edge_scatter = pl.kernel(
    _edge_scatter_body,
    out_type=jax.ShapeDtypeStruct((NC, N, D), jnp.float32),
    mesh=plsc.VectorSubcoreMesh(core_axis_name="c", subcore_axis_name="s"),
    scratch_types=[
        pltpu.VMEM((CHUNK,), jnp.int32),
        pltpu.VMEM((CHUNK, D), jnp.float32),
        pltpu.VMEM((CHUNK,), jnp.int32),
        pltpu.VMEM((CHUNK, D), jnp.float32),
        pltpu.VMEM_SHARED((N, D), jnp.float32),
        pltpu.SemaphoreType.DMA,
        pltpu.SemaphoreType.DMA,
    ],
)


_DN = (((1,), (1,)), ((), ()))


def _eh_body(e_ref, we_ref, be_ref, o_ref):
    o_ref[...] = lax.dot_general(e_ref[...], we_ref[...], _DN,
                                 preferred_element_type=jnp.float32) + be_ref[...]


def _tc_eh(e, W_edge, b_edge):
    return pl.pallas_call(
        _eh_body,
        grid=(E // BE,),
        in_specs=[pl.BlockSpec((BE, DE), lambda i: (i, 0)),
                  pl.BlockSpec((H, DE), lambda i: (0, 0)),
                  pl.BlockSpec((1, H), lambda i: (0, 0))],
        out_specs=pl.BlockSpec((BE, H), lambda i: (i, 0)),
        out_shape=jax.ShapeDtypeStruct((E, H), jnp.float32),
    )(e, W_edge, b_edge.reshape(1, H))


def _prep_body(x_ref, wn_ref, bn_ref, pe_ref, h_ref, eh_ref):
    h_ref[...] = lax.dot_general(x_ref[...], wn_ref[...], _DN,
                                 preferred_element_type=jnp.float32) + bn_ref[...]
    eh_ref[...] = pe_ref[0] + pe_ref[1]


def _tc_prep(x, W_node, b_node, pe):
    return pl.pallas_call(
        _prep_body,
        out_shape=[jax.ShapeDtypeStruct((N, H), jnp.float32),
                   jax.ShapeDtypeStruct((N, H), jnp.float32)],
    )(x, W_node, b_node.reshape(1, H), pe)


def _layer_body(p_ref, eh_ref, tin_ref, w1_ref, b1_ref, g1_ref, be1_ref,
                w2_ref, b2_ref, hs_ref, tout_ref):
    aggr = p_ref[0] + p_ref[1] + eh_ref[...]
    z = lax.dot_general(aggr, w1_ref[...], _DN,
                        preferred_element_type=jnp.float32) + b1_ref[...]
    m = jnp.mean(z, axis=0, keepdims=True)
    v = jnp.mean((z - m) ** 2, axis=0, keepdims=True)
    z = g1_ref[...] * (z - m) / jnp.sqrt(v + 1e-5) + be1_ref[...]
    z = jnp.maximum(z, 0.0)
    z = lax.dot_general(z, w2_ref[...], _DN,
                        preferred_element_type=jnp.float32) + b2_ref[...]
    hs = jnp.where(z > 0, z, 0.2 * z)
    hs_ref[...] = hs
    tout_ref[...] = tin_ref[...] + hs


def _tc_layer(p, eh, total, W1l, b1l, g1l, be1l, W2l, b2l):
    return pl.pallas_call(
        _layer_body,
        out_shape=[jax.ShapeDtypeStruct((N, H), jnp.float32),
                   jax.ShapeDtypeStruct((N, H), jnp.float32)],
    )(p, eh, total, W1l, b1l.reshape(1, 2 * H), g1l.reshape(1, 2 * H),
      be1l.reshape(1, 2 * H), W2l, b2l.reshape(1, H))


def _head_body(t_ref, wo1_ref, bo1_ref, go_ref, beo_ref, ap_ref, wo2_ref,
               bo2_ref, o_ref):
    z = lax.dot_general(t_ref[...], wo1_ref[...], _DN,
                        preferred_element_type=jnp.float32) + bo1_ref[...]
    m = jnp.mean(z, axis=0, keepdims=True)
    v = jnp.mean((z - m) ** 2, axis=0, keepdims=True)
    z = go_ref[...] * (z - m) / jnp.sqrt(v + 1e-5) + beo_ref[...]
    z = jnp.where(z > 0, z, ap_ref[...] * z)
    o_ref[...] = lax.dot_general(z, wo2_ref[...], _DN,
                                 preferred_element_type=jnp.float32) + bo2_ref[...]


def _tc_head(total, Wo1, bo1, go, beo, a_prelu, Wo2, bo2):
    return pl.pallas_call(
        _head_body,
        out_shape=jax.ShapeDtypeStruct((N, C), jnp.float32),
    )(total, Wo1, bo1.reshape(1, 2 * H), go.reshape(1, 2 * H),
      beo.reshape(1, 2 * H), a_prelu.reshape(1, 1), Wo2, bo2.reshape(1, C))


def kernel(x, edge_index, e, W_node, b_node, W_edge, b_edge, W1, b1, g1, be1,
           W2, b2, Wo1, bo1, go, beo, a_prelu, Wo2, bo2):
    src = edge_index[0]
    dst = edge_index[1]

    pad = EPAD - E
    src2 = jnp.concatenate([src, jnp.zeros((pad,), jnp.int32)]).reshape(
        NCHUNKS2, CHUNK)
    dst2 = jnp.concatenate(
        [dst, N + (jnp.arange(pad, dtype=jnp.int32) % (NPAD - N))]).reshape(
        NCHUNKS2, CHUNK)

    zin_d = jnp.zeros((N, D), jnp.float32)

    eh_full = _tc_eh(e, W_edge, b_edge)
    pe = _sc_edge_scatter(eh_full, dst, zin_d)
    h, eh = _tc_prep(x, W_node, b_node, pe)

    hs = h
    total = h
    for l in range(L):
        p = _sc_spmv(hs, src2, dst2, zin_d)
        hs, total = _tc_layer(p, eh, total, W1[l], b1[l], g1[l], be1[l],
                              W2[l], b2[l])

    return _tc_head(total, Wo1, bo1, go, beo, a_prelu, Wo2, bo2)


# speedup vs baseline: 1.7515x; 1.7514x over previous
import jax
import jax.numpy as jnp
from jax import lax
from jax.experimental import pallas as pl
from jax.experimental.pallas import tpu as pltpu
from jax.experimental.pallas import tpu_sc as plsc

N = 10000
E = 320000
D = 128
DE = 16
H = 128
L = 3
C = 40

NC = 2
NS = 16
NW = NC * NS
CHUNK = 128
NCHUNKS = E // CHUNK
CPW = NCHUNKS // NW
NTAIL = NCHUNKS - CPW * NW
RPT = 624
RTAIL = N - RPT * NS
BE = 8000


def _tile_copy(s, src_ref, dst_ref):
    row0 = s * RPT
    pltpu.sync_copy(src_ref.at[pl.ds(row0, RPT), :],
                    dst_ref.at[pl.ds(row0, RPT), :])

    @pl.when(s == NS - 1)
    def _():
        pltpu.sync_copy(src_ref.at[pl.ds(RPT * NS, RTAIL), :],
                        dst_ref.at[pl.ds(RPT * NS, RTAIL), :])


def _spmv_body(table, src, dst, zinit, out,
               src_v0, dst_v0, rows_v0, src_v1, dst_v1, rows_v1,
               acc, isem0, isem1, gsem0, gsem1):
    c = lax.axis_index("c")
    s = lax.axis_index("s")
    wid = c * NS + s

    _tile_copy(s, zinit, acc)
    plsc.subcore_barrier()

    sv = (src_v0, src_v1)
    dv = (dst_v0, dst_v1)
    rv = (rows_v0, rows_v1)
    isems = (isem0, isem1)
    gsems = (gsem0, gsem1)

    def idx_start(j, b):
        base = j * CHUNK
        pltpu.async_copy(src.at[pl.ds(base, CHUNK)], sv[b], isems[b])
        pltpu.async_copy(dst.at[pl.ds(base, CHUNK)], dv[b], isems[b])

    def idx_wait(j, b):
        base = j * CHUNK
        pltpu.make_async_copy(src.at[pl.ds(base, CHUNK)], sv[b], isems[b]).wait()
        pltpu.make_async_copy(dst.at[pl.ds(base, CHUNK)], dv[b], isems[b]).wait()

    def gather_start(b):
        pltpu.async_copy(table.at[sv[b]], rv[b], gsems[b])

    def step(j, b, b1):
        pltpu.make_async_copy(table.at[sv[b]], rv[b], gsems[b]).wait()
        pltpu.sync_copy(rv[b], acc.at[dv[b]], add=True)
        idx_start(j + 2, b)
        idx_wait(j + 1, b1)
        gather_start(b1)

    j0 = wid * CPW
    idx_start(j0, 0)
    idx_start(j0 + 1, 1)
    idx_wait(j0, 0)
    gather_start(0)

    def body(i, _):
        j = j0 + 2 * i
        step(j, 0, 1)
        step(j + 1, 1, 0)
        return 0

    lax.fori_loop(0, (CPW - 2) // 2, body, 0)

    jlast = j0 + CPW - 2
    pltpu.make_async_copy(table.at[sv[0]], rv[0], gsems[0]).wait()
    pltpu.sync_copy(rv[0], acc.at[dv[0]], add=True)
    idx_wait(jlast + 1, 1)
    gather_start(1)
    pltpu.make_async_copy(table.at[sv[1]], rv[1], gsems[1]).wait()
    pltpu.sync_copy(rv[1], acc.at[dv[1]], add=True)

    @pl.when(wid < NTAIL)
    def _():
        jt = NW * CPW + wid
        idx_start(jt, 0)
        idx_wait(jt, 0)
        gather_start(0)
        pltpu.make_async_copy(table.at[sv[0]], rv[0], gsems[0]).wait()
        pltpu.sync_copy(rv[0], acc.at[dv[0]], add=True)

    plsc.subcore_barrier()
    _tile_copy(s, acc, out.at[c])


_sc_spmv = pl.kernel(
    _spmv_body,
    out_type=jax.ShapeDtypeStruct((NC, N, D), jnp.float32),
    mesh=plsc.VectorSubcoreMesh(core_axis_name="c", subcore_axis_name="s"),
    scratch_types=[
        pltpu.VMEM((CHUNK,), jnp.int32),
        pltpu.VMEM((CHUNK,), jnp.int32),
        pltpu.VMEM((CHUNK, D), jnp.float32),
        pltpu.VMEM((CHUNK,), jnp.int32),
        pltpu.VMEM((CHUNK,), jnp.int32),
        pltpu.VMEM((CHUNK, D), jnp.float32),
        pltpu.VMEM_SHARED((N, D), jnp.float32),
        pltpu.SemaphoreType.DMA,
        pltpu.SemaphoreType.DMA,
        pltpu.SemaphoreType.DMA,
        pltpu.SemaphoreType.DMA,
    ],
)


def _edge_scatter_body(eh_full, dst, zinit, out,
                       dst_v0, rows_v0, dst_v1, rows_v1, acc, sem0, sem1):
    c = lax.axis_index("c")
    s = lax.axis_index("s")
    wid = c * NS + s

    _tile_copy(s, zinit, acc)
    plsc.subcore_barrier()

    dv = (dst_v0, dst_v1)
    rv = (rows_v0, rows_v1)
    sems = (sem0, sem1)

    def start(j, b):
        base = j * CHUNK
        pltpu.sync_copy(dst.at[pl.ds(base, CHUNK)], dv[b])
        pltpu.async_copy(eh_full.at[pl.ds(base, CHUNK), :], rv[b], sems[b])

    def finish(j, b):
        base = j * CHUNK
        pltpu.make_async_copy(eh_full.at[pl.ds(base, CHUNK), :], rv[b],
                              sems[b]).wait()
        pltpu.sync_copy(rv[b], acc.at[dv[b]], add=True)

    j0 = wid * CPW
    start(j0, 0)
    start(j0 + 1, 1)

    def body(i, _):
        j = j0 + 2 * i
        finish(j, 0)
        start(j + 2, 0)
        finish(j + 1, 1)
        start(j + 3, 1)
        return 0

    lax.fori_loop(0, (CPW - 2) // 2, body, 0)
    finish(j0 + CPW - 2, 0)
    finish(j0 + CPW - 1, 1)

    @pl.when(wid < NTAIL)
    def _():
        start(NW * CPW + wid, 0)
        finish(NW * CPW + wid, 0)

    plsc.subcore_barrier()
    _tile_copy(s, acc, out.at[c])


_sc_edge_scatter = pl.kernel(
    _edge_scatter_body,
    out_type=jax.ShapeDtypeStruct((NC, N, D), jnp.float32),
    mesh=plsc.VectorSubcoreMesh(core_axis_name="c", subcore_axis_name="s"),
    scratch_types=[
        pltpu.VMEM((CHUNK,), jnp.int32),
        pltpu.VMEM((CHUNK, D), jnp.float32),
        pltpu.VMEM((CHUNK,), jnp.int32),
        pltpu.VMEM((CHUNK, D), jnp.float32),
        pltpu.VMEM_SHARED((N, D), jnp.float32),
        pltpu.SemaphoreType.DMA,
        pltpu.SemaphoreType.DMA,
    ],
)


_DN = (((1,), (1,)), ((), ()))


def _eh_body(e_ref, we_ref, be_ref, o_ref):
    o_ref[...] = lax.dot_general(e_ref[...], we_ref[...], _DN,
                                 preferred_element_type=jnp.float32) + be_ref[...]


def _tc_eh(e, W_edge, b_edge):
    return pl.pallas_call(
        _eh_body,
        grid=(E // BE,),
        in_specs=[pl.BlockSpec((BE, DE), lambda i: (i, 0)),
                  pl.BlockSpec((H, DE), lambda i: (0, 0)),
                  pl.BlockSpec((1, H), lambda i: (0, 0))],
        out_specs=pl.BlockSpec((BE, H), lambda i: (i, 0)),
        out_shape=jax.ShapeDtypeStruct((E, H), jnp.float32),
    )(e, W_edge, b_edge.reshape(1, H))


def _prep_body(x_ref, wn_ref, bn_ref, pe_ref, h_ref, eh_ref):
    h_ref[...] = lax.dot_general(x_ref[...], wn_ref[...], _DN,
                                 preferred_element_type=jnp.float32) + bn_ref[...]
    eh_ref[...] = pe_ref[0] + pe_ref[1]


def _tc_prep(x, W_node, b_node, pe):
    return pl.pallas_call(
        _prep_body,
        out_shape=[jax.ShapeDtypeStruct((N, H), jnp.float32),
                   jax.ShapeDtypeStruct((N, H), jnp.float32)],
    )(x, W_node, b_node.reshape(1, H), pe)


def _layer_body(p_ref, eh_ref, tin_ref, w1_ref, b1_ref, g1_ref, be1_ref,
                w2_ref, b2_ref, hs_ref, tout_ref):
    aggr = p_ref[0] + p_ref[1] + eh_ref[...]
    z = lax.dot_general(aggr, w1_ref[...], _DN,
                        preferred_element_type=jnp.float32) + b1_ref[...]
    m = jnp.mean(z, axis=0, keepdims=True)
    v = jnp.mean((z - m) ** 2, axis=0, keepdims=True)
    z = g1_ref[...] * (z - m) / jnp.sqrt(v + 1e-5) + be1_ref[...]
    z = jnp.maximum(z, 0.0)
    z = lax.dot_general(z, w2_ref[...], _DN,
                        preferred_element_type=jnp.float32) + b2_ref[...]
    hs = jnp.where(z > 0, z, 0.2 * z)
    hs_ref[...] = hs
    tout_ref[...] = tin_ref[...] + hs


def _tc_layer(p, eh, total, W1l, b1l, g1l, be1l, W2l, b2l):
    return pl.pallas_call(
        _layer_body,
        out_shape=[jax.ShapeDtypeStruct((N, H), jnp.float32),
                   jax.ShapeDtypeStruct((N, H), jnp.float32)],
    )(p, eh, total, W1l, b1l.reshape(1, 2 * H), g1l.reshape(1, 2 * H),
      be1l.reshape(1, 2 * H), W2l, b2l.reshape(1, H))


def _head_body(t_ref, wo1_ref, bo1_ref, go_ref, beo_ref, ap_ref, wo2_ref,
               bo2_ref, o_ref):
    z = lax.dot_general(t_ref[...], wo1_ref[...], _DN,
                        preferred_element_type=jnp.float32) + bo1_ref[...]
    m = jnp.mean(z, axis=0, keepdims=True)
    v = jnp.mean((z - m) ** 2, axis=0, keepdims=True)
    z = go_ref[...] * (z - m) / jnp.sqrt(v + 1e-5) + beo_ref[...]
    z = jnp.where(z > 0, z, ap_ref[...] * z)
    o_ref[...] = lax.dot_general(z, wo2_ref[...], _DN,
                                 preferred_element_type=jnp.float32) + bo2_ref[...]


def _tc_head(total, Wo1, bo1, go, beo, a_prelu, Wo2, bo2):
    return pl.pallas_call(
        _head_body,
        out_shape=jax.ShapeDtypeStruct((N, C), jnp.float32),
    )(total, Wo1, bo1.reshape(1, 2 * H), go.reshape(1, 2 * H),
      beo.reshape(1, 2 * H), a_prelu.reshape(1, 1), Wo2, bo2.reshape(1, C))


def kernel(x, edge_index, e, W_node, b_node, W_edge, b_edge, W1, b1, g1, be1,
           W2, b2, Wo1, bo1, go, beo, a_prelu, Wo2, bo2):
    src = edge_index[0]
    dst = edge_index[1]

    zin_d = jnp.zeros((N, D), jnp.float32)

    eh_full = _tc_eh(e, W_edge, b_edge)
    pe = _sc_edge_scatter(eh_full, dst, zin_d)
    h, eh = _tc_prep(x, W_node, b_node, pe)

    hs = h
    total = h
    for l in range(L):
        p = _sc_spmv(hs, src, dst, zin_d)
        hs, total = _tc_layer(p, eh, total, W1[l], b1[l], g1[l], be1[l],
                              W2[l], b2[l])

    return _tc_head(total, Wo1, bo1, go, beo, a_prelu, Wo2, bo2)


# speedup vs baseline: 2.1672x; 1.2373x over previous
import jax
import jax.numpy as jnp
from jax import lax
from jax.experimental import pallas as pl
from jax.experimental.pallas import tpu as pltpu
from jax.experimental.pallas import tpu_sc as plsc

N = 10000
E = 320000
D = 128
DE = 16
H = 128
L = 3
C = 40

NC = 2
NS = 16
NW = NC * NS
CHUNK = 128
NCHUNKS = E // CHUNK
CPW = NCHUNKS // NW
NTAIL = NCHUNKS - CPW * NW
RPT = 624
RTAIL = N - RPT * NS
BE = 8000


def _tile_copy(s, src_ref, dst_ref):
    row0 = s * RPT
    pltpu.sync_copy(src_ref.at[pl.ds(row0, RPT), :],
                    dst_ref.at[pl.ds(row0, RPT), :])

    @pl.when(s == NS - 1)
    def _():
        pltpu.sync_copy(src_ref.at[pl.ds(RPT * NS, RTAIL), :],
                        dst_ref.at[pl.ds(RPT * NS, RTAIL), :])


def _spmv_body(table, src, dst, zinit, out,
               src_v0, dst_v0, rows_v0, src_v1, dst_v1, rows_v1,
               acc, sem0, sem1):
    c = lax.axis_index("c")
    s = lax.axis_index("s")
    wid = c * NS + s

    _tile_copy(s, zinit, acc)
    plsc.subcore_barrier()

    sv = (src_v0, src_v1)
    dv = (dst_v0, dst_v1)
    rv = (rows_v0, rows_v1)
    sems = (sem0, sem1)

    def start(j, b):
        base = j * CHUNK
        pltpu.sync_copy(src.at[pl.ds(base, CHUNK)], sv[b])
        pltpu.sync_copy(dst.at[pl.ds(base, CHUNK)], dv[b])
        pltpu.async_copy(table.at[sv[b]], rv[b], sems[b])

    def finish(b):
        pltpu.make_async_copy(table.at[sv[b]], rv[b], sems[b]).wait()
        pltpu.sync_copy(rv[b], acc.at[dv[b]], add=True)

    j0 = wid * CPW
    start(j0, 0)
    start(j0 + 1, 1)

    def body(i, _):
        j = j0 + 2 * i
        finish(0)
        start(j + 2, 0)
        finish(1)
        start(j + 3, 1)
        return 0

    lax.fori_loop(0, (CPW - 2) // 2, body, 0)
    finish(0)
    finish(1)

    @pl.when(wid < NTAIL)
    def _():
        start(NW * CPW + wid, 0)
        finish(0)

    plsc.subcore_barrier()
    _tile_copy(s, acc, out.at[c])


_sc_spmv = pl.kernel(
    _spmv_body,
    out_type=jax.ShapeDtypeStruct((NC, N, D), jnp.float32),
    mesh=plsc.VectorSubcoreMesh(core_axis_name="c", subcore_axis_name="s"),
    scratch_types=[
        pltpu.VMEM((CHUNK,), jnp.int32),
        pltpu.VMEM((CHUNK,), jnp.int32),
        pltpu.VMEM((CHUNK, D), jnp.float32),
        pltpu.VMEM((CHUNK,), jnp.int32),
        pltpu.VMEM((CHUNK,), jnp.int32),
        pltpu.VMEM((CHUNK, D), jnp.float32),
        pltpu.VMEM_SHARED((N, D), jnp.float32),
        pltpu.SemaphoreType.DMA,
        pltpu.SemaphoreType.DMA,
    ],
)


def _edge_scatter_body(eh_full, dst, zinit, out,
                       dst_v0, rows_v0, dst_v1, rows_v1, acc, sem0, sem1):
    c = lax.axis_index("c")
    s = lax.axis_index("s")
    wid = c * NS + s

    _tile_copy(s, zinit, acc)
    plsc.subcore_barrier()

    dv = (dst_v0, dst_v1)
    rv = (rows_v0, rows_v1)
    sems = (sem0, sem1)

    def start(j, b):
        base = j * CHUNK
        pltpu.sync_copy(dst.at[pl.ds(base, CHUNK)], dv[b])
        pltpu.async_copy(eh_full.at[pl.ds(base, CHUNK), :], rv[b], sems[b])

    def finish(j, b):
        base = j * CHUNK
        pltpu.make_async_copy(eh_full.at[pl.ds(base, CHUNK), :], rv[b],
                              sems[b]).wait()
        pltpu.sync_copy(rv[b], acc.at[dv[b]], add=True)

    j0 = wid * CPW
    start(j0, 0)
    start(j0 + 1, 1)

    def body(i, _):
        j = j0 + 2 * i
        finish(j, 0)
        start(j + 2, 0)
        finish(j + 1, 1)
        start(j + 3, 1)
        return 0

    lax.fori_loop(0, (CPW - 2) // 2, body, 0)
    finish(j0 + CPW - 2, 0)
    finish(j0 + CPW - 1, 1)

    @pl.when(wid < NTAIL)
    def _():
        start(NW * CPW + wid, 0)
        finish(NW * CPW + wid, 0)

    plsc.subcore_barrier()
    _tile_copy(s, acc, out.at[c])


_sc_edge_scatter = pl.kernel(
    _edge_scatter_body,
    out_type=jax.ShapeDtypeStruct((NC, N, D), jnp.float32),
    mesh=plsc.VectorSubcoreMesh(core_axis_name="c", subcore_axis_name="s"),
    scratch_types=[
        pltpu.VMEM((CHUNK,), jnp.int32),
        pltpu.VMEM((CHUNK, D), jnp.float32),
        pltpu.VMEM((CHUNK,), jnp.int32),
        pltpu.VMEM((CHUNK, D), jnp.float32),
        pltpu.VMEM_SHARED((N, D), jnp.float32),
        pltpu.SemaphoreType.DMA,
        pltpu.SemaphoreType.DMA,
    ],
)


_DN = (((1,), (1,)), ((), ()))


def _eh_body(e_ref, we_ref, be_ref, o_ref):
    o_ref[...] = lax.dot_general(e_ref[...], we_ref[...], _DN,
                                 preferred_element_type=jnp.float32) + be_ref[...]


def _tc_eh(e, W_edge, b_edge):
    return pl.pallas_call(
        _eh_body,
        grid=(E // BE,),
        in_specs=[pl.BlockSpec((BE, DE), lambda i: (i, 0)),
                  pl.BlockSpec((H, DE), lambda i: (0, 0)),
                  pl.BlockSpec((1, H), lambda i: (0, 0))],
        out_specs=pl.BlockSpec((BE, H), lambda i: (i, 0)),
        out_shape=jax.ShapeDtypeStruct((E, H), jnp.float32),
    )(e, W_edge, b_edge.reshape(1, H))


def _h_body(x_ref, wn_ref, bn_ref, h_ref):
    h_ref[...] = lax.dot_general(x_ref[...], wn_ref[...], _DN,
                                 preferred_element_type=jnp.float32) + bn_ref[...]


def _tc_h(x, W_node, b_node):
    return pl.pallas_call(
        _h_body,
        out_shape=jax.ShapeDtypeStruct((N, H), jnp.float32),
    )(x, W_node, b_node.reshape(1, H))


def _layer0_body(p_ref, pe_ref, tin_ref, w1_ref, b1_ref, g1_ref, be1_ref,
                 w2_ref, b2_ref, hs_ref, tout_ref, eh_ref):
    eh = pe_ref[0] + pe_ref[1]
    eh_ref[...] = eh
    aggr = p_ref[0] + p_ref[1] + eh
    z = lax.dot_general(aggr, w1_ref[...], _DN,
                        preferred_element_type=jnp.float32) + b1_ref[...]
    m = jnp.mean(z, axis=0, keepdims=True)
    v = jnp.mean((z - m) ** 2, axis=0, keepdims=True)
    z = g1_ref[...] * (z - m) / jnp.sqrt(v + 1e-5) + be1_ref[...]
    z = jnp.maximum(z, 0.0)
    z = lax.dot_general(z, w2_ref[...], _DN,
                        preferred_element_type=jnp.float32) + b2_ref[...]
    hs = jnp.where(z > 0, z, 0.2 * z)
    hs_ref[...] = hs
    tout_ref[...] = tin_ref[...] + hs


def _tc_layer0(p, pe, total, W1l, b1l, g1l, be1l, W2l, b2l):
    return pl.pallas_call(
        _layer0_body,
        out_shape=[jax.ShapeDtypeStruct((N, H), jnp.float32),
                   jax.ShapeDtypeStruct((N, H), jnp.float32),
                   jax.ShapeDtypeStruct((N, H), jnp.float32)],
    )(p, pe, total, W1l, b1l.reshape(1, 2 * H), g1l.reshape(1, 2 * H),
      be1l.reshape(1, 2 * H), W2l, b2l.reshape(1, H))


def _layer_body(p_ref, eh_ref, tin_ref, w1_ref, b1_ref, g1_ref, be1_ref,
                w2_ref, b2_ref, hs_ref, tout_ref):
    aggr = p_ref[0] + p_ref[1] + eh_ref[...]
    z = lax.dot_general(aggr, w1_ref[...], _DN,
                        preferred_element_type=jnp.float32) + b1_ref[...]
    m = jnp.mean(z, axis=0, keepdims=True)
    v = jnp.mean((z - m) ** 2, axis=0, keepdims=True)
    z = g1_ref[...] * (z - m) / jnp.sqrt(v + 1e-5) + be1_ref[...]
    z = jnp.maximum(z, 0.0)
    z = lax.dot_general(z, w2_ref[...], _DN,
                        preferred_element_type=jnp.float32) + b2_ref[...]
    hs = jnp.where(z > 0, z, 0.2 * z)
    hs_ref[...] = hs
    tout_ref[...] = tin_ref[...] + hs


def _tc_layer(p, eh, total, W1l, b1l, g1l, be1l, W2l, b2l):
    return pl.pallas_call(
        _layer_body,
        out_shape=[jax.ShapeDtypeStruct((N, H), jnp.float32),
                   jax.ShapeDtypeStruct((N, H), jnp.float32)],
    )(p, eh, total, W1l, b1l.reshape(1, 2 * H), g1l.reshape(1, 2 * H),
      be1l.reshape(1, 2 * H), W2l, b2l.reshape(1, H))


def _head_body(t_ref, wo1_ref, bo1_ref, go_ref, beo_ref, ap_ref, wo2_ref,
               bo2_ref, o_ref):
    z = lax.dot_general(t_ref[...], wo1_ref[...], _DN,
                        preferred_element_type=jnp.float32) + bo1_ref[...]
    m = jnp.mean(z, axis=0, keepdims=True)
    v = jnp.mean((z - m) ** 2, axis=0, keepdims=True)
    z = go_ref[...] * (z - m) / jnp.sqrt(v + 1e-5) + beo_ref[...]
    z = jnp.where(z > 0, z, ap_ref[...] * z)
    o_ref[...] = lax.dot_general(z, wo2_ref[...], _DN,
                                 preferred_element_type=jnp.float32) + bo2_ref[...]


def _tc_head(total, Wo1, bo1, go, beo, a_prelu, Wo2, bo2):
    return pl.pallas_call(
        _head_body,
        out_shape=jax.ShapeDtypeStruct((N, C), jnp.float32),
    )(total, Wo1, bo1.reshape(1, 2 * H), go.reshape(1, 2 * H),
      beo.reshape(1, 2 * H), a_prelu.reshape(1, 1), Wo2, bo2.reshape(1, C))


def kernel(x, edge_index, e, W_node, b_node, W_edge, b_edge, W1, b1, g1, be1,
           W2, b2, Wo1, bo1, go, beo, a_prelu, Wo2, bo2):
    src = edge_index[0]
    dst = edge_index[1]

    zin_d = jnp.zeros((N, D), jnp.float32)

    h = _tc_h(x, W_node, b_node)
    p = _sc_spmv(h, src, dst, zin_d)
    eh_full = _tc_eh(e, W_edge, b_edge)
    pe = _sc_edge_scatter(eh_full, dst, zin_d)

    hs, total, eh = _tc_layer0(p, pe, h, W1[0], b1[0], g1[0], be1[0],
                               W2[0], b2[0])
    for l in range(1, L):
        p = _sc_spmv(hs, src, dst, zin_d)
        hs, total = _tc_layer(p, eh, total, W1[l], b1[l], g1[l], be1[l],
                              W2[l], b2[l])

    return _tc_head(total, Wo1, bo1, go, beo, a_prelu, Wo2, bo2)
